# baseline (speedup 1.0000x reference)
import functools

import jax
import jax.numpy as jnp
from jax import lax
from jax.experimental import pallas as pl
from jax.experimental.pallas import tpu as pltpu
from jax.experimental.pallas import tpu_sc as plsc

B = 16384
EMB = 32


def _sc_gather(name_table, job_table, names, jobs):
    info = plsc.get_sparse_core_info()
    nc, ns = info.num_cores, info.num_subcores
    nw = nc * ns
    b_per_w = B // nw
    mesh = plsc.VectorSubcoreMesh(core_axis_name="c", subcore_axis_name="s")

    @functools.partial(
        pl.kernel,
        mesh=mesh,
        out_type=(
            jax.ShapeDtypeStruct((B, EMB), jnp.float32),
            jax.ShapeDtypeStruct((B, EMB), jnp.float32),
        ),
        scratch_types=[
            pltpu.VMEM((b_per_w,), jnp.int32),
            pltpu.VMEM((b_per_w,), jnp.int32),
            pltpu.VMEM((b_per_w, EMB), jnp.float32),
            pltpu.VMEM((b_per_w, EMB), jnp.float32),
            pltpu.SemaphoreType.DMA,
            pltpu.SemaphoreType.DMA,
        ],
        compiler_params=pltpu.CompilerParams(use_tc_tiling_on_sc=False),
    )
    def gather_k(ntab, jtab, nidx, jidx, nout, jout,
                 nidx_v, jidx_v, nrows_v, jrows_v, nsem, jsem):
        wid = lax.axis_index("s") * nc + lax.axis_index("c")
        base = wid * b_per_w
        pltpu.sync_copy(nidx.at[pl.ds(base, b_per_w)], nidx_v)
        pltpu.sync_copy(jidx.at[pl.ds(base, b_per_w)], jidx_v)
        ncp = pltpu.async_copy(ntab.at[nidx_v], nrows_v, nsem)
        jcp = pltpu.async_copy(jtab.at[jidx_v], jrows_v, jsem)
        ncp.wait()
        jcp.wait()
        pltpu.sync_copy(nrows_v, nout.at[pl.ds(base, b_per_w)])
        pltpu.sync_copy(jrows_v, jout.at[pl.ds(base, b_per_w)])

    return gather_k(name_table, job_table, names, jobs)


def _mlp_body(ne, je, gd, w1n, w1j, w1g, b1, w2, b2, w3, b3, w4, b4, out):
    x1 = jnp.dot(ne[...], w1n[...], preferred_element_type=jnp.float32)
    x2 = jnp.dot(je[...], w1j[...], preferred_element_type=jnp.float32)
    x3 = jnp.dot(gd[...], w1g[...], preferred_element_type=jnp.float32)
    h = jnp.maximum(x1 + x2 + x3 + b1[...], 0.0)
    h = jnp.maximum(jnp.dot(h, w2[...], preferred_element_type=jnp.float32) + b2[...], 0.0)
    h = jnp.maximum(jnp.dot(h, w3[...], preferred_element_type=jnp.float32) + b3[...], 0.0)
    z = jnp.dot(h, w4[...], preferred_element_type=jnp.float32) + b4[...]
    out[...] = 1.0 / (1.0 + jnp.exp(-z))


def _tc_mlp(ne, je, gd, w1n, w1j, w1g, b1, w2, b2, w3, b3, w4, b4):
    blk = 2048
    grid = (B // blk,)
    full = lambda shape: pl.BlockSpec(shape, lambda i: (0, 0))
    return pl.pallas_call(
        _mlp_body,
        grid=grid,
        in_specs=[
            pl.BlockSpec((blk, EMB), lambda i: (i, 0)),
            pl.BlockSpec((blk, EMB), lambda i: (i, 0)),
            pl.BlockSpec((blk, 4), lambda i: (i, 0)),
            full(w1n.shape), full(w1j.shape), full(w1g.shape), full(b1.shape),
            full(w2.shape), full(b2.shape), full(w3.shape), full(b3.shape),
            full(w4.shape), full(b4.shape),
        ],
        out_specs=pl.BlockSpec((blk, 1), lambda i: (i, 0)),
        out_shape=jax.ShapeDtypeStruct((B, 1), jnp.float32),
    )(ne, je, gd, w1n, w1j, w1g, b1, w2, b2, w3, b3, w4, b4)


def kernel(names, jobs, gender, dob, name_table, job_table,
           W1, b1, W2, b2, W3, b3, W4, b4):
    ne, je = _sc_gather(name_table, job_table, names, jobs)
    gd = jnp.concatenate([gender[:, None], dob], axis=1)
    w1n = W1[:EMB]
    w1j = W1[EMB:2 * EMB]
    w1g = W1[2 * EMB:]
    return _tc_mlp(ne, je, gd,
                   w1n, w1j, w1g, b1[None, :],
                   W2, b2[None, :], W3, b3[None, :], W4, b4[None, :])


# speedup vs baseline: 2.1143x; 2.1143x over previous
import functools

import jax
import jax.numpy as jnp
from jax import lax
from jax.experimental import pallas as pl
from jax.experimental.pallas import tpu as pltpu
from jax.experimental.pallas import tpu_sc as plsc

B = 16384
EMB = 32
PACK = 4
GRP = 8


def _sc_gather(ntab_t, jtab_t, names, jobs):
    info = plsc.get_sparse_core_info()
    nc, ns = info.num_cores, info.num_subcores
    nw = nc * ns
    b_per_w = B // nw
    rows_w = b_per_w // PACK
    n_grp = b_per_w // GRP
    mesh = plsc.VectorSubcoreMesh(core_axis_name="c", subcore_axis_name="s")

    @functools.partial(
        pl.kernel,
        mesh=mesh,
        out_type=jax.ShapeDtypeStruct((B // PACK, 128), jnp.float32),
        scratch_types=[
            pltpu.VMEM((b_per_w,), jnp.int32),
            pltpu.VMEM((GRP * EMB, 128), jnp.float32),
            pltpu.VMEM((GRP * EMB, 128), jnp.float32),
            pltpu.VMEM((rows_w, 128), jnp.float32),
            pltpu.SemaphoreType.DMA,
            pltpu.SemaphoreType.DMA,
        ],
        compiler_params=pltpu.CompilerParams(needs_layout_passes=False),
    )
    def gather_k(tab, idx, out, idx_v, buf0, buf1, out_v, sem0, sem1):
        wid = lax.axis_index("s") * nc + lax.axis_index("c")
        base = wid * b_per_w
        pltpu.sync_copy(idx.at[pl.ds(base, b_per_w)], idx_v)
        row_lo = lax.iota(jnp.int32, 16)
        row_hi = row_lo + 16

        def fire(g, parity, buf, sem):
            vec = idx_v[pl.ds((g // 2) * 16, 16)]
            for c in range(GRP):
                i = vec[parity * GRP + c]
                col0 = pl.multiple_of((i >> 7) * 128, 128)
                pltpu.async_copy(
                    tab.at[:, pl.ds(col0, 128)],
                    buf.at[pl.ds(c * EMB, EMB)], sem)

        def extract(g, parity, buf):
            vec = idx_v[pl.ds((g // 2) * 16, 16)]
            for c in range(GRP):
                i = vec[parity * GRP + c]
                lane = jnp.broadcast_to(i & jnp.int32(127), (16,))
                vlo = plsc.load_gather(buf, [row_lo + c * EMB, lane])
                vhi = plsc.load_gather(buf, [row_hi + c * EMB, lane])
                prow = g * (GRP // PACK) + c // PACK
                pcol = (c % PACK) * EMB
                out_v[prow, pl.ds(pcol, 16)] = vlo
                out_v[prow, pl.ds(pcol + 16, 16)] = vhi

        def drain(sem, buf):
            for _ in range(GRP):
                pltpu.make_async_copy(
                    tab.at[:, pl.ds(0, 128)], buf.at[pl.ds(0, EMB)], sem
                ).wait()

        fire(0, 0, buf0, sem0)

        def body(k, _):
            @pl.when(jnp.logical_and(k + 1 < n_grp, (k + 1) % 2 == 1))
            def _():
                fire(k + 1, 1, buf1, sem1)

            @pl.when(jnp.logical_and(k + 1 < n_grp, (k + 1) % 2 == 0))
            def _():
                fire(k + 1, 0, buf0, sem0)

            @pl.when(k % 2 == 0)
            def _():
                drain(sem0, buf0)
                extract(k, 0, buf0)

            @pl.when(k % 2 == 1)
            def _():
                drain(sem1, buf1)
                extract(k, 1, buf1)

            return ()

        lax.fori_loop(0, n_grp, body, ())
        pltpu.sync_copy(out_v, out.at[pl.ds(wid * rows_w, rows_w)])

    nout = gather_k(ntab_t, names)
    jout = gather_k(jtab_t, jobs)
    return nout, jout


def _mlp_body(ne, je, gd, w1n, w1j, w1g, b1, w2, b2, w3, b3, w4, b4, out):
    x1 = jnp.dot(ne[...], w1n[...], preferred_element_type=jnp.float32)
    x2 = jnp.dot(je[...], w1j[...], preferred_element_type=jnp.float32)
    x3 = jnp.dot(gd[...], w1g[...], preferred_element_type=jnp.float32)
    h = jnp.maximum(x1 + x2 + x3 + b1[...], 0.0)
    h = jnp.maximum(jnp.dot(h, w2[...], preferred_element_type=jnp.float32) + b2[...], 0.0)
    h = jnp.maximum(jnp.dot(h, w3[...], preferred_element_type=jnp.float32) + b3[...], 0.0)
    z = jnp.dot(h, w4[...], preferred_element_type=jnp.float32) + b4[...]
    out[...] = 1.0 / (1.0 + jnp.exp(-z))


def _block_diag(w):
    k, n = w.shape
    out = jnp.zeros((PACK * k, PACK * n), dtype=w.dtype)
    for c in range(PACK):
        out = lax.dynamic_update_slice(out, w, (c * k, c * n))
    return out


def _tc_mlp(ne, je, gd, w1n, w1j, w1g, b1, w2, b2, w3, b3, w4, b4):
    rows = B // PACK
    blk = 512
    grid = (rows // blk,)
    full = lambda a: pl.BlockSpec(a.shape, lambda i: (0, 0))
    return pl.pallas_call(
        _mlp_body,
        grid=grid,
        in_specs=[
            pl.BlockSpec((blk, 128), lambda i: (i, 0)),
            pl.BlockSpec((blk, 128), lambda i: (i, 0)),
            pl.BlockSpec((blk, PACK * 4), lambda i: (i, 0)),
            full(w1n), full(w1j), full(w1g), full(b1),
            full(w2), full(b2), full(w3), full(b3),
            full(w4), full(b4),
        ],
        out_specs=pl.BlockSpec((blk, PACK), lambda i: (i, 0)),
        out_shape=jax.ShapeDtypeStruct((rows, PACK), jnp.float32),
    )(ne, je, gd, w1n, w1j, w1g, b1, w2, b2, w3, b3, w4, b4)


def kernel(names, jobs, gender, dob, name_table, job_table,
           W1, b1, W2, b2, W3, b3, W4, b4):
    ne, je = _sc_gather(name_table.T, job_table.T, names, jobs)
    gd = jnp.concatenate([gender[:, None], dob], axis=1).reshape(B // PACK, PACK * 4)
    w1n = _block_diag(W1[:EMB])
    w1j = _block_diag(W1[EMB:2 * EMB])
    w1g = _block_diag(W1[2 * EMB:])
    tile = lambda b: jnp.tile(b, (PACK,))[None, :]
    out_packed = _tc_mlp(ne, je, gd,
                         w1n, w1j, w1g, tile(b1),
                         _block_diag(W2), tile(b2),
                         _block_diag(W3), tile(b3),
                         _block_diag(W4), tile(b4))
    return out_packed.reshape(B, 1)


# speedup vs baseline: 2.1765x; 1.0294x over previous
import functools

import jax
import jax.numpy as jnp
from jax import lax
from jax.experimental import pallas as pl
from jax.experimental.pallas import tpu as pltpu
from jax.experimental.pallas import tpu_sc as plsc

B = 16384
EMB = 32
PACK = 4
GRP = 4


def _sc_gather(ntab_t, jtab_t, names, jobs):
    info = plsc.get_sparse_core_info()
    nc, ns = info.num_cores, info.num_subcores
    nw = nc * ns
    b_per_w = B // nw
    rows_w = b_per_w // PACK
    n_grp = b_per_w // GRP
    mesh = plsc.VectorSubcoreMesh(core_axis_name="c", subcore_axis_name="s")

    @functools.partial(
        pl.kernel,
        mesh=mesh,
        out_type=(
            jax.ShapeDtypeStruct((B // PACK, 128), jnp.float32),
            jax.ShapeDtypeStruct((B // PACK, 128), jnp.float32),
        ),
        scratch_types=[
            pltpu.VMEM((b_per_w,), jnp.int32),
            pltpu.VMEM((b_per_w,), jnp.int32),
            pltpu.VMEM((GRP * EMB, 128), jnp.float32),
            pltpu.VMEM((GRP * EMB, 128), jnp.float32),
            pltpu.VMEM((GRP * EMB, 128), jnp.float32),
            pltpu.VMEM((GRP * EMB, 128), jnp.float32),
            pltpu.VMEM((rows_w, 128), jnp.float32),
            pltpu.VMEM((rows_w, 128), jnp.float32),
            pltpu.SemaphoreType.DMA,
            pltpu.SemaphoreType.DMA,
        ],
        compiler_params=pltpu.CompilerParams(needs_layout_passes=False),
    )
    def gather_k(ntab, jtab, nidx, jidx, nout, jout,
                 nidx_v, jidx_v, nbuf0, nbuf1, jbuf0, jbuf1,
                 nout_v, jout_v, sem0, sem1):
        wid = lax.axis_index("s") * nc + lax.axis_index("c")
        base = wid * b_per_w
        pltpu.sync_copy(nidx.at[pl.ds(base, b_per_w)], nidx_v)
        pltpu.sync_copy(jidx.at[pl.ds(base, b_per_w)], jidx_v)
        row_lo = lax.iota(jnp.int32, 16)
        row_hi = row_lo + 16

        def fire(g, q, nbuf, jbuf, sem):
            nvec = nidx_v[pl.ds((g // 4) * 16, 16)]
            jvec = jidx_v[pl.ds((g // 4) * 16, 16)]
            for c in range(GRP):
                ni = nvec[q * GRP + c]
                ji = jvec[q * GRP + c]
                ncol = pl.multiple_of((ni >> 7) * 128, 128)
                jcol = pl.multiple_of((ji >> 7) * 128, 128)
                pltpu.async_copy(ntab.at[:, pl.ds(ncol, 128)],
                                 nbuf.at[pl.ds(c * EMB, EMB)], sem)
                pltpu.async_copy(jtab.at[:, pl.ds(jcol, 128)],
                                 jbuf.at[pl.ds(c * EMB, EMB)], sem)

        def extract(g, q, nbuf, jbuf):
            nvec = nidx_v[pl.ds((g // 4) * 16, 16)]
            jvec = jidx_v[pl.ds((g // 4) * 16, 16)]
            for c in range(GRP):
                ni = nvec[q * GRP + c]
                ji = jvec[q * GRP + c]
                nlane = jnp.broadcast_to(ni & jnp.int32(127), (16,))
                jlane = jnp.broadcast_to(ji & jnp.int32(127), (16,))
                pcol = c * EMB
                nout_v[g, pl.ds(pcol, 16)] = plsc.load_gather(
                    nbuf, [row_lo + c * EMB, nlane])
                nout_v[g, pl.ds(pcol + 16, 16)] = plsc.load_gather(
                    nbuf, [row_hi + c * EMB, nlane])
                jout_v[g, pl.ds(pcol, 16)] = plsc.load_gather(
                    jbuf, [row_lo + c * EMB, jlane])
                jout_v[g, pl.ds(pcol + 16, 16)] = plsc.load_gather(
                    jbuf, [row_hi + c * EMB, jlane])

        def drain(sem, nbuf):
            for _ in range(2 * GRP):
                pltpu.make_async_copy(
                    ntab.at[:, pl.ds(0, 128)], nbuf.at[pl.ds(0, EMB)], sem
                ).wait()

        fire(0, 0, nbuf0, jbuf0, sem0)

        def body(k, _):
            g1 = k + 1
            for q in range(4):
                @pl.when(jnp.logical_and(g1 < n_grp, g1 % 4 == q))
                def _(q=q):
                    if q % 2 == 0:
                        fire(g1, q, nbuf0, jbuf0, sem0)
                    else:
                        fire(g1, q, nbuf1, jbuf1, sem1)

            for q in range(4):
                @pl.when(k % 4 == q)
                def _(q=q):
                    if q % 2 == 0:
                        drain(sem0, nbuf0)
                        extract(k, q, nbuf0, jbuf0)
                    else:
                        drain(sem1, nbuf1)
                        extract(k, q, nbuf1, jbuf1)

            return ()

        lax.fori_loop(0, n_grp, body, ())
        pltpu.sync_copy(nout_v, nout.at[pl.ds(wid * rows_w, rows_w)])
        pltpu.sync_copy(jout_v, jout.at[pl.ds(wid * rows_w, rows_w)])

    return gather_k(ntab_t, jtab_t, names, jobs)


def _mlp_body(ne, je, gd, w1n, w1j, w1g, b1, w2, b2, w3, b3, w4, b4, out):
    x1 = jnp.dot(ne[...], w1n[...], preferred_element_type=jnp.float32)
    x2 = jnp.dot(je[...], w1j[...], preferred_element_type=jnp.float32)
    x3 = jnp.dot(gd[...], w1g[...], preferred_element_type=jnp.float32)
    h = jnp.maximum(x1 + x2 + x3 + b1[...], 0.0)
    h = jnp.maximum(jnp.dot(h, w2[...], preferred_element_type=jnp.float32) + b2[...], 0.0)
    h = jnp.maximum(jnp.dot(h, w3[...], preferred_element_type=jnp.float32) + b3[...], 0.0)
    z = jnp.dot(h, w4[...], preferred_element_type=jnp.float32) + b4[...]
    out[...] = 1.0 / (1.0 + jnp.exp(-z))


def _block_diag(w):
    k, n = w.shape
    out = jnp.zeros((PACK * k, PACK * n), dtype=w.dtype)
    for c in range(PACK):
        out = lax.dynamic_update_slice(out, w, (c * k, c * n))
    return out


def _tc_mlp(ne, je, gd, w1n, w1j, w1g, b1, w2, b2, w3, b3, w4, b4):
    rows = B // PACK
    blk = 512
    grid = (rows // blk,)
    full = lambda a: pl.BlockSpec(a.shape, lambda i: (0, 0))
    return pl.pallas_call(
        _mlp_body,
        grid=grid,
        in_specs=[
            pl.BlockSpec((blk, 128), lambda i: (i, 0)),
            pl.BlockSpec((blk, 128), lambda i: (i, 0)),
            pl.BlockSpec((blk, PACK * 4), lambda i: (i, 0)),
            full(w1n), full(w1j), full(w1g), full(b1),
            full(w2), full(b2), full(w3), full(b3),
            full(w4), full(b4),
        ],
        out_specs=pl.BlockSpec((blk, PACK), lambda i: (i, 0)),
        out_shape=jax.ShapeDtypeStruct((rows, PACK), jnp.float32),
    )(ne, je, gd, w1n, w1j, w1g, b1, w2, b2, w3, b3, w4, b4)


def kernel(names, jobs, gender, dob, name_table, job_table,
           W1, b1, W2, b2, W3, b3, W4, b4):
    ne, je = _sc_gather(name_table.T, job_table.T, names, jobs)
    gd = jnp.concatenate([gender[:, None], dob], axis=1).reshape(B // PACK, PACK * 4)
    w1n = _block_diag(W1[:EMB])
    w1j = _block_diag(W1[EMB:2 * EMB])
    w1g = _block_diag(W1[2 * EMB:])
    tile = lambda b: jnp.tile(b, (PACK,))[None, :]
    out_packed = _tc_mlp(ne, je, gd,
                         w1n, w1j, w1g, tile(b1),
                         _block_diag(W2), tile(b2),
                         _block_diag(W3), tile(b3),
                         _block_diag(W4), tile(b4))
    return out_packed.reshape(B, 1)
